# baseline (speedup 1.0000x reference)
import jax
import jax.numpy as jnp
from jax.experimental import pallas as pl


def _decomp_body(x_ref, season_ref, trend_ref):
    xv = x_ref[...]
    row0 = (jax.lax.broadcasted_iota(jnp.int32, xv.shape, 0) == 0) & (
        pl.program_id(0) == 0
    )
    zero = jnp.zeros((), xv.dtype)
    season_ref[...] = jnp.where(row0, zero, xv)
    trend_ref[...] = jnp.where(row0, xv, zero)


def kernel(x):
    n, m = x.shape
    blk = 32
    spec = pl.BlockSpec((blk, m), lambda i: (i, 0))
    season, trend = pl.pallas_call(
        _decomp_body,
        grid=(n // blk,),
        in_specs=[spec],
        out_specs=(spec, spec),
        out_shape=(
            jax.ShapeDtypeStruct((n, m), x.dtype),
            jax.ShapeDtypeStruct((n, m), x.dtype),
        ),
    )(x)
    return (season, trend)


# speedup vs baseline: 1.1410x; 1.1410x over previous
import jax
import jax.numpy as jnp
from jax.experimental import pallas as pl


def _decomp_body(x_ref, season_ref, trend_ref):
    xv = x_ref[...]
    row0 = (jax.lax.broadcasted_iota(jnp.int32, xv.shape, 0) == 0) & (
        pl.program_id(0) == 0
    )
    zero = jnp.zeros((), xv.dtype)
    season_ref[...] = jnp.where(row0, zero, xv)
    trend_ref[...] = jnp.where(row0, xv, zero)


def kernel(x):
    n, m = x.shape
    blk = 64
    spec = pl.BlockSpec((blk, m), lambda i: (i, 0))
    season, trend = pl.pallas_call(
        _decomp_body,
        grid=(n // blk,),
        in_specs=[spec],
        out_specs=(spec, spec),
        out_shape=(
            jax.ShapeDtypeStruct((n, m), x.dtype),
            jax.ShapeDtypeStruct((n, m), x.dtype),
        ),
    )(x)
    return (season, trend)
